# Initial kernel scaffold; baseline (speedup 1.0000x reference)
import jax
import jax.numpy as jnp
from jax.experimental import pallas as pl


def kernel(node_feature, edge_weight, index, mention_count, relation_label, is_train, relation_weight):
    raise NotImplementedError("write your pallas kernel here")


# speedup vs baseline: 1.3963x; 1.3963x over previous
import functools

import jax
import jax.numpy as jnp
from jax.experimental import pallas as pl
from jax.experimental.pallas import tpu as pltpu

B, N, D, R = 8, 4096, 1024, 100
RP = 128
TN = 1024
NT = N // TN


def _flash_kernel(n_ref, edge_ref, q_ref, out_ref, acc_ref, m_ref, s_ref):
    t = pl.program_id(1)

    @pl.when(t == 0)
    def _init():
        m_ref[...] = jnp.full((1, RP), -jnp.inf, jnp.float32)
        s_ref[...] = jnp.zeros((1, RP), jnp.float32)
        acc_ref[...] = jnp.zeros((RP, D), jnp.float32)

    n = n_ref[0]
    q = q_ref[...]
    logits = jax.lax.dot_general(
        n, q, (((1,), (1,)), ((), ())),
        preferred_element_type=jnp.float32)
    mask = edge_ref[0] == 1
    logits = jnp.where(mask, logits, jnp.float32(-1e30))

    m_old = m_ref[...]
    m_new = jnp.maximum(m_old, jnp.max(logits, axis=0, keepdims=True))
    alpha = jnp.exp(m_old - m_new)
    e = jnp.exp(logits - m_new)
    s_ref[...] = s_ref[...] * alpha + jnp.sum(e, axis=0, keepdims=True)
    ev = jax.lax.dot_general(
        e, n, (((0,), (0,)), ((), ())),
        preferred_element_type=jnp.float32)
    acc_ref[...] = acc_ref[...] * alpha.T + ev
    m_ref[...] = m_new

    @pl.when(t == NT - 1)
    def _finish():
        out_ref[0] = acc_ref[...] / s_ref[...].T


@jax.jit
def _run(node_feature, edge_weight, q_pad):
    edge3 = edge_weight.reshape(B * NT, TN, 1)
    out = pl.pallas_call(
        _flash_kernel,
        grid=(B, NT),
        in_specs=[
            pl.BlockSpec((1, TN, D), lambda b, t: (b, t, 0)),
            pl.BlockSpec((1, TN, 1), lambda b, t: (b * NT + t, 0, 0)),
            pl.BlockSpec((RP, D), lambda b, t: (0, 0)),
        ],
        out_specs=pl.BlockSpec((1, RP, D), lambda b, t: (b, 0, 0)),
        out_shape=jax.ShapeDtypeStruct((B, RP, D), jnp.float32),
        scratch_shapes=[
            pltpu.VMEM((RP, D), jnp.float32),
            pltpu.VMEM((1, RP), jnp.float32),
            pltpu.VMEM((1, RP), jnp.float32),
        ],
        compiler_params=pltpu.CompilerParams(
            dimension_semantics=("parallel", "arbitrary"),
        ),
    )(node_feature, edge3, q_pad)
    return out[:, :R, :]


def kernel(node_feature, edge_weight, index, mention_count, relation_label,
           is_train, relation_weight):
    q_pad = jnp.zeros((RP, D), jnp.float32).at[:R].set(relation_weight)
    return _run(node_feature, edge_weight, q_pad)


# speedup vs baseline: 1.4070x; 1.0076x over previous
import functools

import jax
import jax.numpy as jnp
from jax.experimental import pallas as pl
from jax.experimental.pallas import tpu as pltpu

B, N, D, R = 8, 4096, 1024, 100
RP = 128
TN = 1024
NT = N // TN


def _flash_kernel(n_ref, edge_ref, q_ref, out_ref, acc_ref, m_ref, s_ref):
    t = pl.program_id(1)

    @pl.when(t == 0)
    def _init():
        m_ref[...] = jnp.full((1, RP), -jnp.inf, jnp.float32)
        s_ref[...] = jnp.zeros((1, RP), jnp.float32)
        acc_ref[...] = jnp.zeros((RP, D), jnp.float32)

    n = n_ref[0]
    nb = n.astype(jnp.bfloat16)
    q = q_ref[...]
    logits = jax.lax.dot_general(
        nb, q, (((1,), (1,)), ((), ())),
        preferred_element_type=jnp.float32)
    mask = edge_ref[0] == 1
    logits = jnp.where(mask, logits, jnp.float32(-1e30))

    m_old = m_ref[...]
    m_new = jnp.maximum(m_old, jnp.max(logits, axis=0, keepdims=True))
    alpha = jnp.exp(m_old - m_new)
    e = jnp.exp(logits - m_new)
    s_ref[...] = s_ref[...] * alpha + jnp.sum(e, axis=0, keepdims=True)
    ev = jax.lax.dot_general(
        e.astype(jnp.bfloat16), nb, (((0,), (0,)), ((), ())),
        preferred_element_type=jnp.float32)
    acc_ref[...] = acc_ref[...] * alpha.T + ev
    m_ref[...] = m_new

    @pl.when(t == NT - 1)
    def _finish():
        out_ref[0] = acc_ref[...] / s_ref[...].T


@jax.jit
def _run(node_feature, edge_weight, q_pad):
    edge3 = edge_weight.reshape(B * NT, TN, 1)
    out = pl.pallas_call(
        _flash_kernel,
        grid=(B, NT),
        in_specs=[
            pl.BlockSpec((1, TN, D), lambda b, t: (b, t, 0)),
            pl.BlockSpec((1, TN, 1), lambda b, t: (b * NT + t, 0, 0)),
            pl.BlockSpec((RP, D), lambda b, t: (0, 0)),
        ],
        out_specs=pl.BlockSpec((1, RP, D), lambda b, t: (b, 0, 0)),
        out_shape=jax.ShapeDtypeStruct((B, RP, D), jnp.float32),
        scratch_shapes=[
            pltpu.VMEM((RP, D), jnp.float32),
            pltpu.VMEM((1, RP), jnp.float32),
            pltpu.VMEM((1, RP), jnp.float32),
        ],
        compiler_params=pltpu.CompilerParams(
            dimension_semantics=("parallel", "arbitrary"),
        ),
    )(node_feature, edge3, q_pad)
    return out[:, :R, :]


def kernel(node_feature, edge_weight, index, mention_count, relation_label,
           is_train, relation_weight):
    q_pad = jnp.zeros((RP, D), jnp.float32).at[:R].set(relation_weight)
    q_pad = q_pad.astype(jnp.bfloat16)
    return _run(node_feature, edge_weight, q_pad)


# speedup vs baseline: 1.5445x; 1.0977x over previous
import functools

import jax
import jax.numpy as jnp
from jax.experimental import pallas as pl
from jax.experimental.pallas import tpu as pltpu

B, N, D, R = 8, 4096, 1024, 100
RP = 128
TN = 2048
NT = N // TN


def _flash_kernel(n_ref, edge_ref, q_ref, out_ref, acc_ref, m_ref, s_ref):
    t = pl.program_id(1)

    @pl.when(t == 0)
    def _init():
        m_ref[...] = jnp.full((1, RP), -jnp.inf, jnp.float32)
        s_ref[...] = jnp.zeros((1, RP), jnp.float32)
        acc_ref[...] = jnp.zeros((RP, D), jnp.float32)

    n = n_ref[0]
    nb = n.astype(jnp.bfloat16)
    q = q_ref[...]
    logits = jax.lax.dot_general(
        nb, q, (((1,), (1,)), ((), ())),
        preferred_element_type=jnp.float32)
    mask = edge_ref[0] == 1
    logits = jnp.where(mask, logits, jnp.float32(-1e30))

    m_old = m_ref[...]
    m_new = jnp.maximum(m_old, jnp.max(logits, axis=0, keepdims=True))
    alpha = jnp.exp(m_old - m_new)
    e = jnp.exp(logits - m_new)
    s_ref[...] = s_ref[...] * alpha + jnp.sum(e, axis=0, keepdims=True)
    ev = jax.lax.dot_general(
        e.astype(jnp.bfloat16), nb, (((0,), (0,)), ((), ())),
        preferred_element_type=jnp.float32)
    acc_ref[...] = acc_ref[...] * alpha.T + ev
    m_ref[...] = m_new

    @pl.when(t == NT - 1)
    def _finish():
        out_ref[0] = acc_ref[...] / s_ref[...].T


@jax.jit
def _run(node_feature, edge_weight, q_pad):
    edge3 = edge_weight.reshape(B * NT, TN, 1)
    out = pl.pallas_call(
        _flash_kernel,
        grid=(B, NT),
        in_specs=[
            pl.BlockSpec((1, TN, D), lambda b, t: (b, t, 0)),
            pl.BlockSpec((1, TN, 1), lambda b, t: (b * NT + t, 0, 0)),
            pl.BlockSpec((RP, D), lambda b, t: (0, 0)),
        ],
        out_specs=pl.BlockSpec((1, RP, D), lambda b, t: (b, 0, 0)),
        out_shape=jax.ShapeDtypeStruct((B, RP, D), jnp.float32),
        scratch_shapes=[
            pltpu.VMEM((RP, D), jnp.float32),
            pltpu.VMEM((1, RP), jnp.float32),
            pltpu.VMEM((1, RP), jnp.float32),
        ],
        compiler_params=pltpu.CompilerParams(
            dimension_semantics=("parallel", "arbitrary"),
        ),
    )(node_feature, edge3, q_pad)
    return out[:, :R, :]


def kernel(node_feature, edge_weight, index, mention_count, relation_label,
           is_train, relation_weight):
    q_pad = jnp.zeros((RP, D), jnp.float32).at[:R].set(relation_weight)
    q_pad = q_pad.astype(jnp.bfloat16)
    return _run(node_feature, edge_weight, q_pad)


# speedup vs baseline: 1.9633x; 1.2712x over previous
import functools

import jax
import jax.numpy as jnp
from jax.experimental import pallas as pl
from jax.experimental.pallas import tpu as pltpu

B, N, D, R = 8, 4096, 1024, 100
RP = 128
TN = 4096
NT = N // TN


def _flash_kernel(n_ref, edge_ref, q_ref, out_ref, acc_ref, m_ref, s_ref):
    t = pl.program_id(1)

    @pl.when(t == 0)
    def _init():
        m_ref[...] = jnp.full((1, RP), -jnp.inf, jnp.float32)
        s_ref[...] = jnp.zeros((1, RP), jnp.float32)
        acc_ref[...] = jnp.zeros((RP, D), jnp.float32)

    n = n_ref[0]
    nb = n.astype(jnp.bfloat16)
    q = q_ref[...]
    logits = jax.lax.dot_general(
        nb, q, (((1,), (1,)), ((), ())),
        preferred_element_type=jnp.float32)
    mask = edge_ref[0] == 1
    logits = jnp.where(mask, logits, jnp.float32(-1e30))

    m_old = m_ref[...]
    m_new = jnp.maximum(m_old, jnp.max(logits, axis=0, keepdims=True))
    alpha = jnp.exp(m_old - m_new)
    e = jnp.exp(logits - m_new)
    s_ref[...] = s_ref[...] * alpha + jnp.sum(e, axis=0, keepdims=True)
    ev = jax.lax.dot_general(
        e.astype(jnp.bfloat16), nb, (((0,), (0,)), ((), ())),
        preferred_element_type=jnp.float32)
    acc_ref[...] = acc_ref[...] * alpha.T + ev
    m_ref[...] = m_new

    @pl.when(t == NT - 1)
    def _finish():
        out_ref[0] = acc_ref[...] / s_ref[...].T


@jax.jit
def _run(node_feature, edge_weight, q_pad):
    edge3 = edge_weight.reshape(B * NT, TN, 1)
    out = pl.pallas_call(
        _flash_kernel,
        grid=(B, NT),
        in_specs=[
            pl.BlockSpec((1, TN, D), lambda b, t: (b, t, 0)),
            pl.BlockSpec((1, TN, 1), lambda b, t: (b * NT + t, 0, 0)),
            pl.BlockSpec((RP, D), lambda b, t: (0, 0)),
        ],
        out_specs=pl.BlockSpec((1, RP, D), lambda b, t: (b, 0, 0)),
        out_shape=jax.ShapeDtypeStruct((B, RP, D), jnp.float32),
        scratch_shapes=[
            pltpu.VMEM((RP, D), jnp.float32),
            pltpu.VMEM((1, RP), jnp.float32),
            pltpu.VMEM((1, RP), jnp.float32),
        ],
        compiler_params=pltpu.CompilerParams(
            dimension_semantics=("parallel", "arbitrary"),
        ),
    )(node_feature, edge3, q_pad)
    return out[:, :R, :]


def kernel(node_feature, edge_weight, index, mention_count, relation_label,
           is_train, relation_weight):
    q_pad = jnp.zeros((RP, D), jnp.float32).at[:R].set(relation_weight)
    q_pad = q_pad.astype(jnp.bfloat16)
    return _run(node_feature, edge_weight, q_pad)


# speedup vs baseline: 2.0645x; 1.0516x over previous
import jax
import jax.numpy as jnp
from jax.experimental import pallas as pl
from jax.experimental.pallas import tpu as pltpu

B, N, D, R = 8, 4096, 1024, 100
RP = 128
NSPLIT = 4
TN = N // NSPLIT


def _flash_kernel(n0_ref, n1_ref, n2_ref, n3_ref, edge_ref, q_ref, out_ref):
    q = q_ref[...]
    nbs = []
    logits = []
    for h, nr in enumerate((n0_ref, n1_ref, n2_ref, n3_ref)):
        nb = nr[0].astype(jnp.bfloat16)
        nbs.append(nb)
        l = jax.lax.dot_general(
            nb, q, (((1,), (1,)), ((), ())),
            preferred_element_type=jnp.float32)
        mask = edge_ref[0, h * TN:(h + 1) * TN] == 1
        logits.append(jnp.where(mask, l, jnp.float32(-1e30)))

    m = jnp.max(logits[0], axis=0, keepdims=True)
    for l in logits[1:]:
        m = jnp.maximum(m, jnp.max(l, axis=0, keepdims=True))

    s = jnp.zeros((1, RP), jnp.float32)
    acc = jnp.zeros((RP, D), jnp.float32)
    for l, nb in zip(logits, nbs):
        e = jnp.exp(l - m)
        s = s + jnp.sum(e, axis=0, keepdims=True)
        acc = acc + jax.lax.dot_general(
            e.astype(jnp.bfloat16), nb, (((0,), (0,)), ((), ())),
            preferred_element_type=jnp.float32)
    out_ref[0] = acc / s.T


@jax.jit
def _run(node_feature, edge_weight, q_pad):
    edge3 = edge_weight.reshape(B, N, 1)
    nspec = [
        pl.BlockSpec((1, TN, D), lambda b, h=h: (b, h, 0)) for h in range(NSPLIT)
    ]
    out = pl.pallas_call(
        _flash_kernel,
        grid=(B,),
        in_specs=nspec + [
            pl.BlockSpec((1, N, 1), lambda b: (b, 0, 0)),
            pl.BlockSpec((RP, D), lambda b: (0, 0)),
        ],
        out_specs=pl.BlockSpec((1, RP, D), lambda b: (b, 0, 0)),
        out_shape=jax.ShapeDtypeStruct((B, RP, D), jnp.float32),
        compiler_params=pltpu.CompilerParams(
            dimension_semantics=("arbitrary",),
        ),
    )(node_feature, node_feature, node_feature, node_feature, edge3, q_pad)
    return out[:, :R, :]


def kernel(node_feature, edge_weight, index, mention_count, relation_label,
           is_train, relation_weight):
    q_pad = jnp.zeros((RP, D), jnp.float32).at[:R].set(relation_weight)
    q_pad = q_pad.astype(jnp.bfloat16)
    return _run(node_feature, edge_weight, q_pad)
